# baseline (speedup 1.0000x reference)
import functools

import jax
import jax.numpy as jnp
from jax import lax
from jax.experimental import pallas as pl
from jax.experimental.pallas import tpu as pltpu
from jax.experimental.pallas import tpu_sc as plsc

NUM_CORES = 2
NUM_SUBCORES = 16
LANES = 16
NUM_WORKERS = NUM_CORES * NUM_SUBCORES

BATCH = 16384
D = 64
PACK = 2 * D
BPW = BATCH // NUM_WORKERS
CHUNK = 128
NCHUNK = BPW // CHUNK
GROUPS = CHUNK // LANES

_mesh = plsc.VectorSubcoreMesh(core_axis_name="c", subcore_axis_name="s")


@functools.partial(
    pl.kernel,
    out_type=jax.ShapeDtypeStruct((BATCH,), jnp.float32),
    mesh=_mesh,
    compiler_params=pltpu.CompilerParams(
        needs_layout_passes=False, use_tc_tiling_on_sc=True),
    scratch_types=[
        pltpu.VMEM((NCHUNK, CHUNK), jnp.int32),
        pltpu.VMEM((NCHUNK, CHUNK), jnp.int32),
        pltpu.VMEM((NCHUNK, CHUNK), jnp.int32),
        pltpu.VMEM((NCHUNK, CHUNK), jnp.int32),
        pltpu.VMEM((2, CHUNK, PACK), jnp.float32),
        pltpu.VMEM((2, CHUNK, PACK), jnp.float32),
        pltpu.VMEM((PACK,), jnp.float32),
        pltpu.VMEM((BPW,), jnp.float32),
        pltpu.SemaphoreType.DMA,
        pltpu.SemaphoreType.DMA,
    ],
)
def _gmf_kernel(users_hbm, movies_hbm, ut_hbm, mt_hbm, w_hbm, out_hbm,
                uidx_v, midx_v, urow_v, mrow_v, urows_v, mrows_v, w_v, out_v,
                sem_a, sem_b):
    wid = lax.axis_index("s") * NUM_CORES + lax.axis_index("c")
    base = wid * BPW

    pltpu.sync_copy(w_hbm, w_v)
    for c in range(NCHUNK):
        pltpu.sync_copy(users_hbm.at[pl.ds(base + c * CHUNK, CHUNK)], uidx_v.at[c])
        pltpu.sync_copy(movies_hbm.at[pl.ds(base + c * CHUNK, CHUNK)], midx_v.at[c])

    for c in range(NCHUNK):
        for v in range(CHUNK // LANES):
            sl = pl.ds(v * LANES, LANES)
            urow_v[c, sl] = lax.shift_right_logical(uidx_v[c, sl], 1)
            mrow_v[c, sl] = lax.shift_right_logical(midx_v[c, sl], 1)

    wvecs = [w_v[pl.ds(k * LANES, LANES)] for k in range(D // LANES)]
    ws = [wvecs[d // LANES][d % LANES] for d in range(D)]

    sems = [sem_a, sem_b]

    def start_gather(c):
        buf = c % 2
        pltpu.async_copy(ut_hbm.at[urow_v.at[c]], urows_v.at[buf], sems[buf])
        pltpu.async_copy(mt_hbm.at[mrow_v.at[c]], mrows_v.at[buf], sems[buf])

    def wait_gather(c):
        buf = c % 2
        pltpu.make_async_copy(ut_hbm.at[urow_v.at[c]], urows_v.at[buf], sems[buf]).wait()
        pltpu.make_async_copy(mt_hbm.at[mrow_v.at[c]], mrows_v.at[buf], sems[buf]).wait()

    lane_iota = lax.iota(jnp.int32, LANES)

    start_gather(0)
    for c in range(NCHUNK):
        if c + 1 < NCHUNK:
            start_gather(c + 1)
        wait_gather(c)
        buf = c % 2
        u_buf = urows_v.at[buf]
        m_buf = mrows_v.at[buf]

        def group_body(g, carry, u_buf=u_buf, m_buf=m_buf, c=c):
            gsl = pl.ds(g * LANES, LANES)
            upar = lax.bitwise_and(uidx_v[c, gsl], 1) * D
            mpar = lax.bitwise_and(midx_v[c, gsl], 1) * D
            rows = lane_iota + g * LANES
            acc = jnp.zeros((LANES,), jnp.float32)
            for d in range(D):
                uv = plsc.load_gather(u_buf, [rows, upar + d])
                mv = plsc.load_gather(m_buf, [rows, mpar + d])
                acc = acc + uv * mv * ws[d]
            out_v[pl.ds(c * CHUNK + g * LANES, LANES)] = acc
            return carry

        lax.fori_loop(0, GROUPS, group_body, 0)

    pltpu.sync_copy(out_v, out_hbm.at[pl.ds(base, BPW)])


def kernel(users, movies, user_table, movie_table, W):
    ut2 = user_table.reshape(user_table.shape[0] // 2, PACK)
    mt2 = movie_table.reshape(movie_table.shape[0] // 2, PACK)
    w2 = jnp.concatenate([W.reshape(D), jnp.zeros((D,), jnp.float32)])
    out = _gmf_kernel(users, movies, ut2, mt2, w2)
    return out.reshape(BATCH, 1)


# speedup vs baseline: 1.6148x; 1.6148x over previous
import functools

import jax
import jax.numpy as jnp
from jax import lax
from jax.experimental import pallas as pl
from jax.experimental.pallas import tpu as pltpu
from jax.experimental.pallas import tpu_sc as plsc

NUM_CORES = 2
NUM_SUBCORES = 16
LANES = 16
NUM_WORKERS = NUM_CORES * NUM_SUBCORES

BATCH = 16384
D = 64
BPW = BATCH // NUM_WORKERS
NVEC = BATCH // LANES
NBKT = 256
HITCAP = 1024
BLK = 128

NUSERS = 1000000
NMOVIES = 100000

_mesh = plsc.VectorSubcoreMesh(core_axis_name="c", subcore_axis_name="s")


def _extract_one(table_hbm, idx_hbm, out_hbm, n_rows,
                 idxbuf_v, cnt_v, cur_v, sorted_v, sorted_s, starts_s,
                 blockbuf_v, stage_v, sem_blk, sem_out, wid):
    max_j = (n_rows - 1) // BLK
    nloc = max_j // NUM_WORKERS + 2

    lane = lax.iota(jnp.int32, LANES)
    zeros16 = jnp.zeros((LANES,), jnp.int32)
    ones16 = jnp.ones((LANES,), jnp.int32)

    pltpu.sync_copy(idx_hbm, idxbuf_v)

    for k in range(NBKT // LANES):
        cnt_v[pl.ds(k * LANES, LANES)] = zeros16

    def count_body(v, carry):
        u = idxbuf_v[pl.ds(v * LANES, LANES)]
        j = lax.shift_right_logical(u, 7)
        own = lax.bitwise_and(j, NUM_WORKERS - 1) == wid
        l = lax.shift_right_logical(j, 5)
        rank, last = plsc.scan_count(l, mask=own)
        plsc.addupdate_scatter(cnt_v, [l], rank,
                               mask=lax.bitwise_and(last, own))
        return carry

    lax.fori_loop(0, NVEC, count_body, 0)

    carry = jnp.int32(0)
    for k in range(NBKT // LANES):
        sl = pl.ds(k * LANES, LANES)
        c = cnt_v[sl]
        cum = plsc.cumsum(c)
        excl = cum - c + carry
        cur_v[sl] = excl
        for j in range(LANES):
            starts_s[k * LANES + j] = excl[j]
        carry = carry + cum[LANES - 1]
    total_hits = carry
    starts_s[NBKT] = total_hits

    def scat_body(v, carry):
        u = idxbuf_v[pl.ds(v * LANES, LANES)]
        j = lax.shift_right_logical(u, 7)
        own = lax.bitwise_and(j, NUM_WORKERS - 1) == wid
        l = lax.shift_right_logical(j, 5)
        rank, last = plsc.scan_count(l, mask=own)
        base = plsc.load_gather(cur_v, [l])
        pos = base + rank - 1
        b_vec = lane + v * LANES
        packed = lax.shift_left(b_vec, 7) + lax.bitwise_and(u, BLK - 1)
        plsc.store_scatter(sorted_v, [pos], packed, mask=own)
        plsc.store_scatter(cur_v, [l], pos + ones16,
                           mask=lax.bitwise_and(last, own))
        return carry

    lax.fori_loop(0, NVEC, scat_body, 0)

    def smem_body(v, carry):
        s = sorted_v[pl.ds(v * LANES, LANES)]
        for j in range(LANES):
            sorted_s[v * LANES + j] = s[j]
        return carry

    nhv = lax.div(total_hits + (LANES - 1), jnp.int32(LANES))
    lax.fori_loop(0, nhv, smem_body, 0)

    def fetch(l):
        j = l * NUM_WORKERS + wid
        par = lax.rem(l, 2)

        @pl.when(jnp.logical_and(j <= max_j, l < nloc))
        def _():
            off = pl.multiple_of(j * BLK, BLK)
            row = pl.multiple_of(par * D, 8)
            pltpu.async_copy(
                table_hbm.at[pl.ds(0, D), pl.ds(off, BLK)],
                blockbuf_v.at[pl.ds(row, D), pl.ds(0, BLK)], sem_blk)

    fetch(jnp.int32(0))

    def sweep_body(l, carry):
        j = l * NUM_WORKERS + wid
        par = lax.rem(l, 2)
        inflight = carry

        @pl.when(j <= max_j)
        def _():
            row = pl.multiple_of(par * D, 8)
            pltpu.make_async_copy(
                table_hbm.at[pl.ds(0, D), pl.ds(0, BLK)],
                blockbuf_v.at[pl.ds(row, D), pl.ds(0, BLK)], sem_blk).wait()

        fetch(l + 1)

        def hit_body(h, infl, par=par):
            s = sorted_s[h]
            b = lax.shift_right_logical(s, 7)
            rl = lax.bitwise_and(s, BLK - 1)
            slot = lax.rem(h, 4)

            @pl.when(infl >= 4)
            def _():
                pltpu.make_async_copy(
                    stage_v.at[pl.ds(0, D)], out_hbm.at[pl.ds(0, D)],
                    sem_out).wait()

            rowbase = lane + par * D
            for k in range(D // LANES):
                col = plsc.load_gather(
                    blockbuf_v,
                    [rowbase + k * LANES,
                     jnp.full((LANES,), 0, jnp.int32) + rl])
                stage_v[pl.ds(slot * D + k * LANES, LANES)] = col
            pltpu.async_copy(stage_v.at[pl.ds(slot * D, D)],
                             out_hbm.at[pl.ds(b * D, D)], sem_out)
            return infl + 1

        lo = starts_s[l]
        hi = jnp.where(l + 1 < NBKT, starts_s[l + 1], total_hits)
        hi = jnp.where(j <= max_j, hi, lo)
        inflight = lax.fori_loop(lo, hi, hit_body, inflight)
        return inflight

    inflight = lax.fori_loop(0, nloc, sweep_body, jnp.int32(0))

    def drain_body(i, carry):
        pltpu.make_async_copy(
            stage_v.at[pl.ds(0, D)], out_hbm.at[pl.ds(0, D)], sem_out).wait()
        return carry

    lax.fori_loop(0, lax.min(inflight, jnp.int32(4)), drain_body, 0)


@functools.partial(
    pl.kernel,
    out_type=(jax.ShapeDtypeStruct((BATCH * D,), jnp.float32),
              jax.ShapeDtypeStruct((BATCH * D,), jnp.float32)),
    mesh=_mesh,
    compiler_params=pltpu.CompilerParams(
        needs_layout_passes=False, use_tc_tiling_on_sc=True),
    scratch_types=[
        pltpu.VMEM((BATCH,), jnp.int32),
        pltpu.VMEM((NBKT,), jnp.int32),
        pltpu.VMEM((NBKT,), jnp.int32),
        pltpu.VMEM((HITCAP,), jnp.int32),
        pltpu.SMEM((HITCAP,), jnp.int32),
        pltpu.SMEM((NBKT + 1,), jnp.int32),
        pltpu.VMEM((2 * D, BLK), jnp.float32),
        pltpu.VMEM((4 * D,), jnp.float32),
        pltpu.SemaphoreType.DMA,
        pltpu.SemaphoreType.DMA,
    ],
)
def _extract_kernel(users_hbm, movies_hbm, utT_hbm, mtT_hbm,
                    uout_hbm, mout_hbm,
                    idxbuf_v, cnt_v, cur_v, sorted_v, sorted_s, starts_s,
                    blockbuf_v, stage_v, sem_blk, sem_out):
    wid = lax.axis_index("s") * NUM_CORES + lax.axis_index("c")
    _extract_one(mtT_hbm, movies_hbm, mout_hbm, NMOVIES,
                 idxbuf_v, cnt_v, cur_v, sorted_v, sorted_s, starts_s,
                 blockbuf_v, stage_v, sem_blk, sem_out, wid)
    _extract_one(utT_hbm, users_hbm, uout_hbm, NUSERS,
                 idxbuf_v, cnt_v, cur_v, sorted_v, sorted_s, starts_s,
                 blockbuf_v, stage_v, sem_blk, sem_out, wid)


CHUNK = 128
NCHUNK = BPW // CHUNK
GROUPS = CHUNK // LANES


@functools.partial(
    pl.kernel,
    out_type=jax.ShapeDtypeStruct((BATCH,), jnp.float32),
    mesh=_mesh,
    compiler_params=pltpu.CompilerParams(needs_layout_passes=False),
    scratch_types=[
        pltpu.VMEM((2 * CHUNK * D,), jnp.float32),
        pltpu.VMEM((2 * CHUNK * D,), jnp.float32),
        pltpu.VMEM((D,), jnp.float32),
        pltpu.VMEM((BPW,), jnp.float32),
        pltpu.SemaphoreType.DMA,
        pltpu.SemaphoreType.DMA,
    ],
)
def _dot_kernel(urows_hbm, mrows_hbm, w_hbm, out_hbm,
                ubuf_v, mbuf_v, w_v, out_v, sem_a, sem_b):
    wid = lax.axis_index("s") * NUM_CORES + lax.axis_index("c")
    base = wid * BPW * D

    pltpu.sync_copy(w_hbm, w_v)
    wvecs = [w_v[pl.ds(k * LANES, LANES)] for k in range(D // LANES)]
    ws = [wvecs[d // LANES][d % LANES] for d in range(D)]

    sems = [sem_a, sem_b]

    def start(c):
        buf = c % 2
        sl = pl.ds(base + c * CHUNK * D, CHUNK * D)
        dst = pl.ds(buf * CHUNK * D, CHUNK * D)
        pltpu.async_copy(urows_hbm.at[sl], ubuf_v.at[dst], sems[buf])
        pltpu.async_copy(mrows_hbm.at[sl], mbuf_v.at[dst], sems[buf])

    def wait(c):
        buf = c % 2
        sl = pl.ds(base + c * CHUNK * D, CHUNK * D)
        dst = pl.ds(buf * CHUNK * D, CHUNK * D)
        pltpu.make_async_copy(urows_hbm.at[sl], ubuf_v.at[dst], sems[buf]).wait()
        pltpu.make_async_copy(mrows_hbm.at[sl], mbuf_v.at[dst], sems[buf]).wait()

    lane64 = lax.iota(jnp.int32, LANES) * D

    start(0)
    for c in range(NCHUNK):
        if c + 1 < NCHUNK:
            start(c + 1)
        wait(c)
        buf = c % 2

        def group_body(g, carry, buf=buf, c=c):
            gbase = lane64 + (g * (LANES * D) + buf * CHUNK * D)
            acc = jnp.zeros((LANES,), jnp.float32)
            for d in range(D):
                idx = gbase + d
                uv = plsc.load_gather(ubuf_v, [idx])
                mv = plsc.load_gather(mbuf_v, [idx])
                acc = acc + uv * mv * ws[d]
            out_v[pl.ds(c * CHUNK + g * LANES, LANES)] = acc
            return carry

        lax.fori_loop(0, GROUPS, group_body, 0)

    pltpu.sync_copy(out_v, out_hbm.at[pl.ds(wid * BPW, BPW)])


def kernel(users, movies, user_table, movie_table, W):
    urows, mrows = _extract_kernel(users, movies, user_table.T, movie_table.T)
    out = _dot_kernel(urows, mrows, W.reshape(D))
    return out.reshape(BATCH, 1)


# speedup vs baseline: 2.4265x; 1.5027x over previous
import functools

import jax
import jax.numpy as jnp
from jax import lax
from jax.experimental import pallas as pl
from jax.experimental.pallas import tpu as pltpu
from jax.experimental.pallas import tpu_sc as plsc

NUM_CORES = 2
NUM_SUBCORES = 16
LANES = 16
NUM_WORKERS = NUM_CORES * NUM_SUBCORES

BATCH = 16384
D = 64
BPW = BATCH // NUM_WORKERS
NVEC = BATCH // LANES
NBKT = 256
HITCAP = 1024
BLK = 128

NUSERS = 1000000
NMOVIES = 100000

_mesh = plsc.VectorSubcoreMesh(core_axis_name="c", subcore_axis_name="s")


def _extract_one(table_hbm, idx_hbm, out_hbm, n_rows,
                 idxbuf_v, cnt_v, cur_v, sorted_v, sorted_s, starts_s,
                 blockbuf_v, stage_v, sem_blk, sem_out, wid):
    max_j = (n_rows - 1) // BLK
    nloc = max_j // NUM_WORKERS + 2

    lane = lax.iota(jnp.int32, LANES)
    zeros16 = jnp.zeros((LANES,), jnp.int32)
    ones16 = jnp.ones((LANES,), jnp.int32)

    pltpu.sync_copy(idx_hbm, idxbuf_v)

    for k in range(NBKT // LANES):
        cnt_v[pl.ds(k * LANES, LANES)] = zeros16

    def count_body(v4, carry):
        for i in range(4):
            u = idxbuf_v[pl.ds((v4 * 4 + i) * LANES, LANES)]
            j = lax.shift_right_logical(u, 7)
            own = lax.bitwise_and(j, NUM_WORKERS - 1) == wid
            l = lax.shift_right_logical(j, 5)
            rank, last = plsc.scan_count(l, mask=own)
            plsc.addupdate_scatter(cnt_v, [l], rank,
                                   mask=lax.bitwise_and(last, own))
        return carry

    lax.fori_loop(0, NVEC // 4, count_body, 0)

    carry = jnp.int32(0)
    for k in range(NBKT // LANES):
        sl = pl.ds(k * LANES, LANES)
        c = cnt_v[sl]
        cum = plsc.cumsum(c)
        excl = cum - c + carry
        cur_v[sl] = excl
        for j in range(LANES):
            starts_s[k * LANES + j] = excl[j]
        carry = carry + cum[LANES - 1]
    total_hits = carry
    starts_s[NBKT] = total_hits

    def scat_body(v4, carry):
        for i in range(4):
            v = v4 * 4 + i
            u = idxbuf_v[pl.ds(v * LANES, LANES)]
            j = lax.shift_right_logical(u, 7)
            own = lax.bitwise_and(j, NUM_WORKERS - 1) == wid
            l = lax.shift_right_logical(j, 5)
            rank, last = plsc.scan_count(l, mask=own)
            base = plsc.load_gather(cur_v, [l])
            pos = base + rank - 1
            b_vec = lane + v * LANES
            packed = lax.shift_left(b_vec, 7) + lax.bitwise_and(u, BLK - 1)
            plsc.store_scatter(sorted_v, [pos], packed, mask=own)
            plsc.store_scatter(cur_v, [l], pos + ones16,
                               mask=lax.bitwise_and(last, own))
        return carry

    lax.fori_loop(0, NVEC // 4, scat_body, 0)

    def smem_body(v, carry):
        s = sorted_v[pl.ds(v * LANES, LANES)]
        for j in range(LANES):
            sorted_s[v * LANES + j] = s[j]
        return carry

    nhv = lax.div(total_hits + (LANES - 1), jnp.int32(LANES))
    lax.fori_loop(0, nhv, smem_body, 0)

    def fetch(l):
        j = l * NUM_WORKERS + wid
        par = lax.rem(l, 4)

        @pl.when(jnp.logical_and(j <= max_j, l < nloc))
        def _():
            off = pl.multiple_of(j * BLK, BLK)
            row = pl.multiple_of(par * D, 8)
            pltpu.async_copy(
                table_hbm.at[pl.ds(0, D), pl.ds(off, BLK)],
                blockbuf_v.at[pl.ds(row, D), pl.ds(0, BLK)], sem_blk)

    fetch(jnp.int32(0))
    fetch(jnp.int32(1))
    fetch(jnp.int32(2))

    def sweep_body(l, carry):
        j = l * NUM_WORKERS + wid
        par = lax.rem(l, 4)
        inflight = carry

        @pl.when(j <= max_j)
        def _():
            row = pl.multiple_of(par * D, 8)
            pltpu.make_async_copy(
                table_hbm.at[pl.ds(0, D), pl.ds(0, BLK)],
                blockbuf_v.at[pl.ds(row, D), pl.ds(0, BLK)], sem_blk).wait()

        fetch(l + 3)

        def hit_body(h, infl, par=par):
            s = sorted_s[h]
            b = lax.shift_right_logical(s, 7)
            rl = lax.bitwise_and(s, BLK - 1)
            slot = lax.rem(h, 16)

            @pl.when(infl >= 16)
            def _():
                pltpu.make_async_copy(
                    stage_v.at[pl.ds(0, D)], out_hbm.at[pl.ds(0, D)],
                    sem_out).wait()

            rowbase = lane + par * D
            for k in range(D // LANES):
                col = plsc.load_gather(
                    blockbuf_v,
                    [rowbase + k * LANES,
                     jnp.full((LANES,), 0, jnp.int32) + rl])
                stage_v[pl.ds(slot * D + k * LANES, LANES)] = col
            pltpu.async_copy(stage_v.at[pl.ds(slot * D, D)],
                             out_hbm.at[pl.ds(b * D, D)], sem_out)
            return infl + 1

        lo = starts_s[l]
        hi = jnp.where(l + 1 < NBKT, starts_s[l + 1], total_hits)
        hi = jnp.where(j <= max_j, hi, lo)
        inflight = lax.fori_loop(lo, hi, hit_body, inflight)
        return inflight

    inflight = lax.fori_loop(0, nloc, sweep_body, jnp.int32(0))

    def drain_body(i, carry):
        pltpu.make_async_copy(
            stage_v.at[pl.ds(0, D)], out_hbm.at[pl.ds(0, D)], sem_out).wait()
        return carry

    lax.fori_loop(0, lax.min(inflight, jnp.int32(16)), drain_body, 0)


@functools.partial(
    pl.kernel,
    out_type=(jax.ShapeDtypeStruct((BATCH * D,), jnp.float32),
              jax.ShapeDtypeStruct((BATCH * D,), jnp.float32)),
    mesh=_mesh,
    compiler_params=pltpu.CompilerParams(
        needs_layout_passes=False, use_tc_tiling_on_sc=True),
    scratch_types=[
        pltpu.VMEM((BATCH,), jnp.int32),
        pltpu.VMEM((NBKT,), jnp.int32),
        pltpu.VMEM((NBKT,), jnp.int32),
        pltpu.VMEM((HITCAP,), jnp.int32),
        pltpu.SMEM((HITCAP,), jnp.int32),
        pltpu.SMEM((NBKT + 1,), jnp.int32),
        pltpu.VMEM((4 * D, BLK), jnp.float32),
        pltpu.VMEM((16 * D,), jnp.float32),
        pltpu.SemaphoreType.DMA,
        pltpu.SemaphoreType.DMA,
    ],
)
def _extract_kernel(users_hbm, movies_hbm, utT_hbm, mtT_hbm,
                    uout_hbm, mout_hbm,
                    idxbuf_v, cnt_v, cur_v, sorted_v, sorted_s, starts_s,
                    blockbuf_v, stage_v, sem_blk, sem_out):
    wid = lax.axis_index("s") * NUM_CORES + lax.axis_index("c")
    _extract_one(mtT_hbm, movies_hbm, mout_hbm, NMOVIES,
                 idxbuf_v, cnt_v, cur_v, sorted_v, sorted_s, starts_s,
                 blockbuf_v, stage_v, sem_blk, sem_out, wid)
    _extract_one(utT_hbm, users_hbm, uout_hbm, NUSERS,
                 idxbuf_v, cnt_v, cur_v, sorted_v, sorted_s, starts_s,
                 blockbuf_v, stage_v, sem_blk, sem_out, wid)


CHUNK = 128
NCHUNK = BPW // CHUNK
GROUPS = CHUNK // LANES


@functools.partial(
    pl.kernel,
    out_type=jax.ShapeDtypeStruct((BATCH,), jnp.float32),
    mesh=_mesh,
    compiler_params=pltpu.CompilerParams(needs_layout_passes=False),
    scratch_types=[
        pltpu.VMEM((2 * CHUNK * D,), jnp.float32),
        pltpu.VMEM((2 * CHUNK * D,), jnp.float32),
        pltpu.VMEM((D,), jnp.float32),
        pltpu.VMEM((BPW,), jnp.float32),
        pltpu.SemaphoreType.DMA,
        pltpu.SemaphoreType.DMA,
    ],
)
def _dot_kernel(urows_hbm, mrows_hbm, w_hbm, out_hbm,
                ubuf_v, mbuf_v, w_v, out_v, sem_a, sem_b):
    wid = lax.axis_index("s") * NUM_CORES + lax.axis_index("c")
    base = wid * BPW * D

    pltpu.sync_copy(w_hbm, w_v)
    wvecs = [w_v[pl.ds(k * LANES, LANES)] for k in range(D // LANES)]
    ws = [wvecs[d // LANES][d % LANES] for d in range(D)]

    sems = [sem_a, sem_b]

    def start(c):
        buf = c % 2
        sl = pl.ds(base + c * CHUNK * D, CHUNK * D)
        dst = pl.ds(buf * CHUNK * D, CHUNK * D)
        pltpu.async_copy(urows_hbm.at[sl], ubuf_v.at[dst], sems[buf])
        pltpu.async_copy(mrows_hbm.at[sl], mbuf_v.at[dst], sems[buf])

    def wait(c):
        buf = c % 2
        sl = pl.ds(base + c * CHUNK * D, CHUNK * D)
        dst = pl.ds(buf * CHUNK * D, CHUNK * D)
        pltpu.make_async_copy(urows_hbm.at[sl], ubuf_v.at[dst], sems[buf]).wait()
        pltpu.make_async_copy(mrows_hbm.at[sl], mbuf_v.at[dst], sems[buf]).wait()

    lane64 = lax.iota(jnp.int32, LANES) * D

    start(0)
    for c in range(NCHUNK):
        if c + 1 < NCHUNK:
            start(c + 1)
        wait(c)
        buf = c % 2

        def group_body(g, carry, buf=buf, c=c):
            gbase = lane64 + (g * (LANES * D) + buf * CHUNK * D)
            acc = jnp.zeros((LANES,), jnp.float32)
            for d in range(D):
                idx = gbase + d
                uv = plsc.load_gather(ubuf_v, [idx])
                mv = plsc.load_gather(mbuf_v, [idx])
                acc = acc + uv * mv * ws[d]
            out_v[pl.ds(c * CHUNK + g * LANES, LANES)] = acc
            return carry

        lax.fori_loop(0, GROUPS, group_body, 0)

    pltpu.sync_copy(out_v, out_hbm.at[pl.ds(wid * BPW, BPW)])


def kernel(users, movies, user_table, movie_table, W):
    urows, mrows = _extract_kernel(users, movies, user_table.T, movie_table.T)
    out = _dot_kernel(urows, mrows, W.reshape(D))
    return out.reshape(BATCH, 1)


# speedup vs baseline: 2.6824x; 1.1054x over previous
import functools

import jax
import jax.numpy as jnp
from jax import lax
from jax.experimental import pallas as pl
from jax.experimental.pallas import tpu as pltpu
from jax.experimental.pallas import tpu_sc as plsc

NUM_CORES = 2
NUM_SUBCORES = 16
LANES = 16
NUM_WORKERS = NUM_CORES * NUM_SUBCORES

BATCH = 16384
D = 64
BPW = BATCH // NUM_WORKERS
NVEC = BATCH // LANES
NBKT = 256
HITCAP = 1024
BLK = 128

NUSERS = 1000000
NMOVIES = 100000

_mesh = plsc.VectorSubcoreMesh(core_axis_name="c", subcore_axis_name="s")


def _extract_one(table_hbm, idx_hbm, out_hbm, n_rows,
                 idxbuf_v, cnt_v, cur_v, sorted_v, sorted_s, starts_s,
                 blockbuf_v, stage_v, sem_blk, sem_out, wid):
    max_j = (n_rows - 1) // BLK
    nloc = max_j // NUM_WORKERS + 2

    lane = lax.iota(jnp.int32, LANES)
    zeros16 = jnp.zeros((LANES,), jnp.int32)
    ones16 = jnp.ones((LANES,), jnp.int32)

    pltpu.sync_copy(idx_hbm, idxbuf_v)

    for k in range(NBKT // LANES):
        cnt_v[pl.ds(k * LANES, LANES)] = zeros16

    def count_body(v4, carry):
        for i in range(4):
            u = idxbuf_v[pl.ds((v4 * 4 + i) * LANES, LANES)]
            j = lax.shift_right_logical(u, 7)
            own = lax.bitwise_and(j, NUM_WORKERS - 1) == wid
            l = lax.shift_right_logical(j, 5)
            rank, last = plsc.scan_count(l, mask=own)
            plsc.addupdate_scatter(cnt_v, [l], rank,
                                   mask=lax.bitwise_and(last, own))
        return carry

    lax.fori_loop(0, NVEC // 4, count_body, 0)

    carry = jnp.int32(0)
    for k in range(NBKT // LANES):
        sl = pl.ds(k * LANES, LANES)
        c = cnt_v[sl]
        cum = plsc.cumsum(c)
        excl = cum - c + carry
        cur_v[sl] = excl
        for j in range(LANES):
            starts_s[k * LANES + j] = excl[j]
        carry = carry + cum[LANES - 1]
    total_hits = carry
    starts_s[NBKT] = total_hits

    def scat_body(v4, carry):
        for i in range(4):
            v = v4 * 4 + i
            u = idxbuf_v[pl.ds(v * LANES, LANES)]
            j = lax.shift_right_logical(u, 7)
            own = lax.bitwise_and(j, NUM_WORKERS - 1) == wid
            l = lax.shift_right_logical(j, 5)
            rank, last = plsc.scan_count(l, mask=own)
            base = plsc.load_gather(cur_v, [l])
            pos = base + rank - 1
            b_vec = lane + v * LANES
            packed = lax.shift_left(b_vec, 7) + lax.bitwise_and(u, BLK - 1)
            plsc.store_scatter(sorted_v, [pos], packed, mask=own)
            plsc.store_scatter(cur_v, [l], pos + ones16,
                               mask=lax.bitwise_and(last, own))
        return carry

    lax.fori_loop(0, NVEC // 4, scat_body, 0)

    def smem_body(v, carry):
        s = sorted_v[pl.ds(v * LANES, LANES)]
        for j in range(LANES):
            sorted_s[v * LANES + j] = s[j]
        return carry

    nhv = lax.div(total_hits + (LANES - 1), jnp.int32(LANES))
    lax.fori_loop(0, nhv, smem_body, 0)

    def nonempty(l):
        lo = starts_s[l]
        hi = jnp.where(l + 1 < NBKT, starts_s[l + 1], total_hits)
        return hi > lo

    def fetch(l):
        j = l * NUM_WORKERS + wid
        par = lax.rem(l, 8)

        @pl.when(jnp.logical_and(
            jnp.logical_and(j <= max_j, l < nloc), nonempty(l)))
        def _():
            off = pl.multiple_of(j * BLK, BLK)
            row = pl.multiple_of(par * D, 8)
            pltpu.async_copy(
                table_hbm.at[pl.ds(0, D), pl.ds(off, BLK)],
                blockbuf_v.at[pl.ds(row, D), pl.ds(0, BLK)], sem_blk)

    for p in range(7):
        fetch(jnp.int32(p))

    def sweep_body(l, carry):
        j = l * NUM_WORKERS + wid
        par = lax.rem(l, 8)
        inflight = carry

        @pl.when(jnp.logical_and(j <= max_j, nonempty(l)))
        def _():
            row = pl.multiple_of(par * D, 8)
            pltpu.make_async_copy(
                table_hbm.at[pl.ds(0, D), pl.ds(0, BLK)],
                blockbuf_v.at[pl.ds(row, D), pl.ds(0, BLK)], sem_blk).wait()

        fetch(l + 7)

        def hit_body(h, infl, par=par):
            s = sorted_s[h]
            b = lax.shift_right_logical(s, 7)
            rl = lax.bitwise_and(s, BLK - 1)
            slot = lax.rem(h, 16)

            @pl.when(infl >= 16)
            def _():
                pltpu.make_async_copy(
                    stage_v.at[pl.ds(0, D)], out_hbm.at[pl.ds(0, D)],
                    sem_out).wait()

            rowbase = lane + par * D
            for k in range(D // LANES):
                col = plsc.load_gather(
                    blockbuf_v,
                    [rowbase + k * LANES,
                     jnp.full((LANES,), 0, jnp.int32) + rl])
                stage_v[pl.ds(slot * D + k * LANES, LANES)] = col
            pltpu.async_copy(stage_v.at[pl.ds(slot * D, D)],
                             out_hbm.at[pl.ds(b * D, D)], sem_out)
            return infl + 1

        lo = starts_s[l]
        hi = jnp.where(l + 1 < NBKT, starts_s[l + 1], total_hits)
        hi = jnp.where(j <= max_j, hi, lo)
        inflight = lax.fori_loop(lo, hi, hit_body, inflight)
        return inflight

    inflight = lax.fori_loop(0, nloc, sweep_body, jnp.int32(0))

    def drain_body(i, carry):
        pltpu.make_async_copy(
            stage_v.at[pl.ds(0, D)], out_hbm.at[pl.ds(0, D)], sem_out).wait()
        return carry

    lax.fori_loop(0, lax.min(inflight, jnp.int32(16)), drain_body, 0)


@functools.partial(
    pl.kernel,
    out_type=(jax.ShapeDtypeStruct((BATCH * D,), jnp.float32),
              jax.ShapeDtypeStruct((BATCH * D,), jnp.float32)),
    mesh=_mesh,
    compiler_params=pltpu.CompilerParams(
        needs_layout_passes=False, use_tc_tiling_on_sc=True),
    scratch_types=[
        pltpu.VMEM((BATCH,), jnp.int32),
        pltpu.VMEM((NBKT,), jnp.int32),
        pltpu.VMEM((NBKT,), jnp.int32),
        pltpu.VMEM((HITCAP,), jnp.int32),
        pltpu.SMEM((HITCAP,), jnp.int32),
        pltpu.SMEM((NBKT + 1,), jnp.int32),
        pltpu.VMEM((8 * D, BLK), jnp.float32),
        pltpu.VMEM((16 * D,), jnp.float32),
        pltpu.SemaphoreType.DMA,
        pltpu.SemaphoreType.DMA,
    ],
)
def _extract_kernel(users_hbm, movies_hbm, utT_hbm, mtT_hbm,
                    uout_hbm, mout_hbm,
                    idxbuf_v, cnt_v, cur_v, sorted_v, sorted_s, starts_s,
                    blockbuf_v, stage_v, sem_blk, sem_out):
    wid = lax.axis_index("s") * NUM_CORES + lax.axis_index("c")
    _extract_one(mtT_hbm, movies_hbm, mout_hbm, NMOVIES,
                 idxbuf_v, cnt_v, cur_v, sorted_v, sorted_s, starts_s,
                 blockbuf_v, stage_v, sem_blk, sem_out, wid)
    _extract_one(utT_hbm, users_hbm, uout_hbm, NUSERS,
                 idxbuf_v, cnt_v, cur_v, sorted_v, sorted_s, starts_s,
                 blockbuf_v, stage_v, sem_blk, sem_out, wid)


CHUNK = 128
NCHUNK = BPW // CHUNK
GROUPS = CHUNK // LANES


@functools.partial(
    pl.kernel,
    out_type=jax.ShapeDtypeStruct((BATCH,), jnp.float32),
    mesh=_mesh,
    compiler_params=pltpu.CompilerParams(needs_layout_passes=False),
    scratch_types=[
        pltpu.VMEM((2 * CHUNK * D,), jnp.float32),
        pltpu.VMEM((2 * CHUNK * D,), jnp.float32),
        pltpu.VMEM((D,), jnp.float32),
        pltpu.VMEM((BPW,), jnp.float32),
        pltpu.SemaphoreType.DMA,
        pltpu.SemaphoreType.DMA,
    ],
)
def _dot_kernel(urows_hbm, mrows_hbm, w_hbm, out_hbm,
                ubuf_v, mbuf_v, w_v, out_v, sem_a, sem_b):
    wid = lax.axis_index("s") * NUM_CORES + lax.axis_index("c")
    base = wid * BPW * D

    pltpu.sync_copy(w_hbm, w_v)
    wvecs = [w_v[pl.ds(k * LANES, LANES)] for k in range(D // LANES)]
    ws = [wvecs[d // LANES][d % LANES] for d in range(D)]

    sems = [sem_a, sem_b]

    def start(c):
        buf = c % 2
        sl = pl.ds(base + c * CHUNK * D, CHUNK * D)
        dst = pl.ds(buf * CHUNK * D, CHUNK * D)
        pltpu.async_copy(urows_hbm.at[sl], ubuf_v.at[dst], sems[buf])
        pltpu.async_copy(mrows_hbm.at[sl], mbuf_v.at[dst], sems[buf])

    def wait(c):
        buf = c % 2
        sl = pl.ds(base + c * CHUNK * D, CHUNK * D)
        dst = pl.ds(buf * CHUNK * D, CHUNK * D)
        pltpu.make_async_copy(urows_hbm.at[sl], ubuf_v.at[dst], sems[buf]).wait()
        pltpu.make_async_copy(mrows_hbm.at[sl], mbuf_v.at[dst], sems[buf]).wait()

    lane64 = lax.iota(jnp.int32, LANES) * D

    start(0)
    for c in range(NCHUNK):
        if c + 1 < NCHUNK:
            start(c + 1)
        wait(c)
        buf = c % 2

        def group_body(g, carry, buf=buf, c=c):
            gbase = lane64 + (g * (LANES * D) + buf * CHUNK * D)
            acc = jnp.zeros((LANES,), jnp.float32)
            for d in range(D):
                idx = gbase + d
                uv = plsc.load_gather(ubuf_v, [idx])
                mv = plsc.load_gather(mbuf_v, [idx])
                acc = acc + uv * mv * ws[d]
            out_v[pl.ds(c * CHUNK + g * LANES, LANES)] = acc
            return carry

        lax.fori_loop(0, GROUPS, group_body, 0)

    pltpu.sync_copy(out_v, out_hbm.at[pl.ds(wid * BPW, BPW)])


def kernel(users, movies, user_table, movie_table, W):
    urows, mrows = _extract_kernel(users, movies, user_table.T, movie_table.T)
    out = _dot_kernel(urows, mrows, W.reshape(D))
    return out.reshape(BATCH, 1)


# speedup vs baseline: 3.0008x; 1.1187x over previous
import functools

import jax
import jax.numpy as jnp
from jax import lax
from jax.experimental import pallas as pl
from jax.experimental.pallas import tpu as pltpu
from jax.experimental.pallas import tpu_sc as plsc

NUM_CORES = 2
NUM_SUBCORES = 16
LANES = 16
NUM_WORKERS = NUM_CORES * NUM_SUBCORES

BATCH = 16384
D = 64
BPW = BATCH // NUM_WORKERS
NVEC = BATCH // LANES
NBKT = 256
HITCAP = 1024
BLK = 128

NUSERS = 1000000
NMOVIES = 100000

_mesh = plsc.VectorSubcoreMesh(core_axis_name="c", subcore_axis_name="s")


def _extract_one(table_hbm, idx_hbm, out_hbm, n_rows,
                 idxbuf_v, cnt_v, cur_v, sorted_v, sorted_s, starts_s,
                 blockbuf_v, stage_v, sem_blk, sem_out, wid):
    max_j = (n_rows - 1) // BLK
    nloc = max_j // NUM_WORKERS + 2

    lane = lax.iota(jnp.int32, LANES)
    zeros16 = jnp.zeros((LANES,), jnp.int32)
    ones16 = jnp.ones((LANES,), jnp.int32)

    pltpu.sync_copy(idx_hbm, idxbuf_v)

    for k in range(NBKT // LANES):
        cnt_v[pl.ds(k * LANES, LANES)] = zeros16

    def count_body(v4, carry):
        for i in range(4):
            u = idxbuf_v[pl.ds((v4 * 4 + i) * LANES, LANES)]
            j = lax.shift_right_logical(u, 7)
            own = lax.bitwise_and(j, NUM_WORKERS - 1) == wid
            l = lax.shift_right_logical(j, 5)
            rank, last = plsc.scan_count(l, mask=own)
            plsc.addupdate_scatter(cnt_v, [l], rank,
                                   mask=lax.bitwise_and(last, own))
        return carry

    lax.fori_loop(0, NVEC // 4, count_body, 0)

    carry = jnp.int32(0)
    for k in range(NBKT // LANES):
        sl = pl.ds(k * LANES, LANES)
        c = cnt_v[sl]
        cum = plsc.cumsum(c)
        excl = cum - c + carry
        cur_v[sl] = excl
        for j in range(LANES):
            starts_s[k * LANES + j] = excl[j]
        carry = carry + cum[LANES - 1]
    total_hits = carry
    starts_s[NBKT] = total_hits

    def scat_body(v4, carry):
        for i in range(4):
            v = v4 * 4 + i
            u = idxbuf_v[pl.ds(v * LANES, LANES)]
            j = lax.shift_right_logical(u, 7)
            own = lax.bitwise_and(j, NUM_WORKERS - 1) == wid
            l = lax.shift_right_logical(j, 5)
            rank, last = plsc.scan_count(l, mask=own)
            base = plsc.load_gather(cur_v, [l])
            pos = base + rank - 1
            b_vec = lane + v * LANES
            packed = lax.shift_left(b_vec, 7) + lax.bitwise_and(u, BLK - 1)
            plsc.store_scatter(sorted_v, [pos], packed, mask=own)
            plsc.store_scatter(cur_v, [l], pos + ones16,
                               mask=lax.bitwise_and(last, own))
        return carry

    lax.fori_loop(0, NVEC // 4, scat_body, 0)

    def smem_body(v, carry):
        s = sorted_v[pl.ds(v * LANES, LANES)]
        for j in range(LANES):
            sorted_s[v * LANES + j] = s[j]
        return carry

    nhv = lax.div(total_hits + (LANES - 1), jnp.int32(LANES))
    lax.fori_loop(0, nhv, smem_body, 0)

    def nonempty(l):
        lo = starts_s[l]
        hi = jnp.where(l + 1 < NBKT, starts_s[l + 1], total_hits)
        return hi > lo

    def fetch(l):
        j = l * NUM_WORKERS + wid
        par = lax.rem(l, 8)

        @pl.when(jnp.logical_and(
            jnp.logical_and(j <= max_j, l < nloc), nonempty(l)))
        def _():
            off = pl.multiple_of(j * BLK, BLK)
            row = pl.multiple_of(par * D, 8)
            pltpu.async_copy(
                table_hbm.at[pl.ds(0, D), pl.ds(off, BLK)],
                blockbuf_v.at[pl.ds(row, D), pl.ds(0, BLK)], sem_blk)

    for p in range(7):
        fetch(jnp.int32(p))

    def sweep_body(l, carry):
        j = l * NUM_WORKERS + wid
        par = lax.rem(l, 8)
        inflight = carry

        @pl.when(jnp.logical_and(j <= max_j, nonempty(l)))
        def _():
            row = pl.multiple_of(par * D, 8)
            pltpu.make_async_copy(
                table_hbm.at[pl.ds(0, D), pl.ds(0, BLK)],
                blockbuf_v.at[pl.ds(row, D), pl.ds(0, BLK)], sem_blk).wait()

        fetch(l + 7)

        def hit_body(h, infl, par=par):
            s = sorted_s[h]
            b = lax.shift_right_logical(s, 7)
            rl = lax.bitwise_and(s, BLK - 1)
            slot = lax.rem(h, 16)

            @pl.when(infl >= 16)
            def _():
                pltpu.make_async_copy(
                    stage_v.at[pl.ds(0, D)], out_hbm.at[pl.ds(0, D)],
                    sem_out).wait()

            rowbase = lane + par * D
            for k in range(D // LANES):
                col = plsc.load_gather(
                    blockbuf_v,
                    [rowbase + k * LANES,
                     jnp.full((LANES,), 0, jnp.int32) + rl])
                stage_v[pl.ds(slot * D + k * LANES, LANES)] = col
            pltpu.async_copy(stage_v.at[pl.ds(slot * D, D)],
                             out_hbm.at[pl.ds(b * D, D)], sem_out)
            return infl + 1

        lo = starts_s[l]
        hi = jnp.where(l + 1 < NBKT, starts_s[l + 1], total_hits)
        hi = jnp.where(j <= max_j, hi, lo)
        inflight = lax.fori_loop(lo, hi, hit_body, inflight)
        return inflight

    inflight = lax.fori_loop(0, nloc, sweep_body, jnp.int32(0))

    def drain_body(i, carry):
        pltpu.make_async_copy(
            stage_v.at[pl.ds(0, D)], out_hbm.at[pl.ds(0, D)], sem_out).wait()
        return carry

    lax.fori_loop(0, lax.min(inflight, jnp.int32(16)), drain_body, 0)


@functools.partial(
    pl.kernel,
    out_type=(jax.ShapeDtypeStruct((BATCH * D,), jnp.float32),
              jax.ShapeDtypeStruct((BATCH * D,), jnp.float32)),
    mesh=_mesh,
    compiler_params=pltpu.CompilerParams(
        needs_layout_passes=False, use_tc_tiling_on_sc=True),
    scratch_types=[
        pltpu.VMEM((BATCH,), jnp.int32),
        pltpu.VMEM((NBKT,), jnp.int32),
        pltpu.VMEM((NBKT,), jnp.int32),
        pltpu.VMEM((HITCAP,), jnp.int32),
        pltpu.SMEM((HITCAP,), jnp.int32),
        pltpu.SMEM((NBKT + 1,), jnp.int32),
        pltpu.VMEM((8 * D, BLK), jnp.float32),
        pltpu.VMEM((16 * D,), jnp.float32),
        pltpu.SemaphoreType.DMA,
        pltpu.SemaphoreType.DMA,
    ],
)
def _extract_kernel(users_hbm, movies_hbm, utT_hbm, mtT_hbm,
                    uout_hbm, mout_hbm,
                    idxbuf_v, cnt_v, cur_v, sorted_v, sorted_s, starts_s,
                    blockbuf_v, stage_v, sem_blk, sem_out):
    wid = lax.axis_index("s") * NUM_CORES + lax.axis_index("c")
    _extract_one(mtT_hbm, movies_hbm, mout_hbm, NMOVIES,
                 idxbuf_v, cnt_v, cur_v, sorted_v, sorted_s, starts_s,
                 blockbuf_v, stage_v, sem_blk, sem_out, wid)
    _extract_one(utT_hbm, users_hbm, uout_hbm, NUSERS,
                 idxbuf_v, cnt_v, cur_v, sorted_v, sorted_s, starts_s,
                 blockbuf_v, stage_v, sem_blk, sem_out, wid)


TC_TILE = 2048


def _tc_dot_body(u_ref, m_ref, w_ref, o_ref):
    x = u_ref[...] * m_ref[...]
    o_ref[...] = jax.lax.dot_general(
        x, w_ref[...], (((1,), (1,)), ((), ())),
        preferred_element_type=jnp.float32)


_tc_dot = pl.pallas_call(
    _tc_dot_body,
    grid=(BATCH // TC_TILE,),
    in_specs=[
        pl.BlockSpec((TC_TILE, D), lambda i: (i, 0)),
        pl.BlockSpec((TC_TILE, D), lambda i: (i, 0)),
        pl.BlockSpec((1, D), lambda i: (0, 0)),
    ],
    out_specs=pl.BlockSpec((TC_TILE, 1), lambda i: (i, 0)),
    out_shape=jax.ShapeDtypeStruct((BATCH, 1), jnp.float32),
)


def kernel(users, movies, user_table, movie_table, W):
    urows, mrows = _extract_kernel(users, movies, user_table.T, movie_table.T)
    return _tc_dot(urows.reshape(BATCH, D), mrows.reshape(BATCH, D), W)


# speedup vs baseline: 3.8286x; 1.2759x over previous
import functools

import jax
import jax.numpy as jnp
from jax import lax
from jax.experimental import pallas as pl
from jax.experimental.pallas import tpu as pltpu
from jax.experimental.pallas import tpu_sc as plsc

NUM_CORES = 2
NUM_SUBCORES = 16
LANES = 16
NUM_WORKERS = NUM_CORES * NUM_SUBCORES

BATCH = 16384
D = 64
BPW = BATCH // NUM_WORKERS
NVEC = BATCH // LANES
NBKT = 256
HITCAP = 1024
BLK = 128

NUSERS = 1000000
NMOVIES = 100000

_mesh = plsc.VectorSubcoreMesh(core_axis_name="c", subcore_axis_name="s")


def _extract_one(table_hbm, idx_hbm, out_hbm, n_rows,
                 idxbuf_v, cnt_v, cur_v, hits_v, sorted_v, sorted_s, starts_s,
                 blockbuf_v, stage_v, sem_blk, sem_out, wid):
    max_j = (n_rows - 1) // BLK
    nloc = max_j // NUM_WORKERS + 2

    lane = lax.iota(jnp.int32, LANES)
    zeros16 = jnp.zeros((LANES,), jnp.int32)
    ones16 = jnp.ones((LANES,), jnp.int32)

    pltpu.sync_copy(idx_hbm, idxbuf_v)

    for k in range(NBKT // LANES):
        cnt_v[pl.ds(k * LANES, LANES)] = zeros16

    def compact_body(v4, cursor):
        for i in range(4):
            v = v4 * 4 + i
            u = idxbuf_v[pl.ds(v * LANES, LANES)]
            j = lax.shift_right_logical(u, 7)
            own = lax.bitwise_and(j, NUM_WORKERS - 1) == wid
            l = lax.shift_right_logical(j, 5)
            rl = lax.bitwise_and(u, BLK - 1)
            b_vec = lane + v * LANES
            packed = lax.bitwise_or(
                lax.bitwise_or(lax.shift_left(b_vec, 15),
                               lax.shift_left(l, 7)), rl)
            plsc.store_compressed(hits_v.at[pl.ds(cursor, LANES)], packed, mask=own)
            cnt = plsc.all_reduce_population_count(own)
            cursor = cursor + cnt[0]
        return cursor

    total_hits = lax.fori_loop(0, NVEC // 4, compact_body, jnp.int32(0))
    nhv = lax.div(total_hits + (LANES - 1), jnp.int32(LANES))

    def count_body(v, carry):
        p = hits_v[pl.ds(v * LANES, LANES)]
        valid = (lane + v * LANES) < total_hits
        l = lax.bitwise_and(lax.shift_right_logical(p, 7), NBKT - 1)
        rank, last = plsc.scan_count(l, mask=valid)
        plsc.addupdate_scatter(cnt_v, [l], rank,
                               mask=lax.bitwise_and(last, valid))
        return carry

    lax.fori_loop(0, nhv, count_body, 0)

    carry = jnp.int32(0)
    for k in range(NBKT // LANES):
        sl = pl.ds(k * LANES, LANES)
        c = cnt_v[sl]
        cum = plsc.cumsum(c)
        excl = cum - c + carry
        cur_v[sl] = excl
        for j in range(LANES):
            starts_s[k * LANES + j] = excl[j]
        carry = carry + cum[LANES - 1]
    starts_s[NBKT] = total_hits

    def scat_body(v, carry):
        p = hits_v[pl.ds(v * LANES, LANES)]
        valid = (lane + v * LANES) < total_hits
        l = lax.bitwise_and(lax.shift_right_logical(p, 7), NBKT - 1)
        rank, last = plsc.scan_count(l, mask=valid)
        base = plsc.load_gather(cur_v, [l])
        pos = base + rank - 1
        b_vec = lax.shift_right_logical(p, 15)
        p2 = lax.bitwise_or(lax.shift_left(b_vec, 7),
                            lax.bitwise_and(p, BLK - 1))
        plsc.store_scatter(sorted_v, [pos], p2, mask=valid)
        plsc.store_scatter(cur_v, [l], pos + ones16,
                           mask=lax.bitwise_and(last, valid))
        return carry

    lax.fori_loop(0, nhv, scat_body, 0)

    def smem_body(v, carry):
        s = sorted_v[pl.ds(v * LANES, LANES)]
        for j in range(LANES):
            sorted_s[v * LANES + j] = s[j]
        return carry

    lax.fori_loop(0, nhv, smem_body, 0)

    def nonempty(l):
        lo = starts_s[l]
        hi = jnp.where(l + 1 < NBKT, starts_s[l + 1], total_hits)
        return hi > lo

    def fetch(l):
        j = l * NUM_WORKERS + wid
        par = lax.rem(l, 8)

        @pl.when(jnp.logical_and(
            jnp.logical_and(j <= max_j, l < nloc), nonempty(l)))
        def _():
            off = pl.multiple_of(j * BLK, BLK)
            row = pl.multiple_of(par * D, 8)
            pltpu.async_copy(
                table_hbm.at[pl.ds(0, D), pl.ds(off, BLK)],
                blockbuf_v.at[pl.ds(row, D), pl.ds(0, BLK)], sem_blk)

    for p in range(7):
        fetch(jnp.int32(p))

    def sweep_body(l, carry):
        j = l * NUM_WORKERS + wid
        par = lax.rem(l, 8)
        inflight = carry

        @pl.when(jnp.logical_and(j <= max_j, nonempty(l)))
        def _():
            row = pl.multiple_of(par * D, 8)
            pltpu.make_async_copy(
                table_hbm.at[pl.ds(0, D), pl.ds(0, BLK)],
                blockbuf_v.at[pl.ds(row, D), pl.ds(0, BLK)], sem_blk).wait()

        fetch(l + 7)

        def hit_body(h, infl, par=par):
            s = sorted_s[h]
            b = lax.shift_right_logical(s, 7)
            rl = lax.bitwise_and(s, BLK - 1)
            slot = lax.rem(h, 16)

            @pl.when(infl >= 16)
            def _():
                pltpu.make_async_copy(
                    stage_v.at[pl.ds(0, D)], out_hbm.at[pl.ds(0, D)],
                    sem_out).wait()

            rowbase = lane + par * D
            for k in range(D // LANES):
                col = plsc.load_gather(
                    blockbuf_v,
                    [rowbase + k * LANES,
                     jnp.full((LANES,), 0, jnp.int32) + rl])
                stage_v[pl.ds(slot * D + k * LANES, LANES)] = col
            pltpu.async_copy(stage_v.at[pl.ds(slot * D, D)],
                             out_hbm.at[pl.ds(b * D, D)], sem_out)
            return infl + 1

        lo = starts_s[l]
        hi = jnp.where(l + 1 < NBKT, starts_s[l + 1], total_hits)
        hi = jnp.where(j <= max_j, hi, lo)
        inflight = lax.fori_loop(lo, hi, hit_body, inflight)
        return inflight

    inflight = lax.fori_loop(0, nloc, sweep_body, jnp.int32(0))

    def drain_body(i, carry):
        pltpu.make_async_copy(
            stage_v.at[pl.ds(0, D)], out_hbm.at[pl.ds(0, D)], sem_out).wait()
        return carry

    lax.fori_loop(0, lax.min(inflight, jnp.int32(16)), drain_body, 0)


@functools.partial(
    pl.kernel,
    out_type=(jax.ShapeDtypeStruct((BATCH * D,), jnp.float32),
              jax.ShapeDtypeStruct((BATCH * D,), jnp.float32)),
    mesh=_mesh,
    compiler_params=pltpu.CompilerParams(
        needs_layout_passes=False, use_tc_tiling_on_sc=True),
    scratch_types=[
        pltpu.VMEM((BATCH,), jnp.int32),
        pltpu.VMEM((NBKT,), jnp.int32),
        pltpu.VMEM((NBKT,), jnp.int32),
        pltpu.VMEM((HITCAP,), jnp.int32),
        pltpu.VMEM((HITCAP,), jnp.int32),
        pltpu.SMEM((HITCAP,), jnp.int32),
        pltpu.SMEM((NBKT + 1,), jnp.int32),
        pltpu.VMEM((8 * D, BLK), jnp.float32),
        pltpu.VMEM((16 * D,), jnp.float32),
        pltpu.SemaphoreType.DMA,
        pltpu.SemaphoreType.DMA,
    ],
)
def _extract_kernel(users_hbm, movies_hbm, utT_hbm, mtT_hbm,
                    uout_hbm, mout_hbm,
                    idxbuf_v, cnt_v, cur_v, hits_v, sorted_v, sorted_s,
                    starts_s, blockbuf_v, stage_v, sem_blk, sem_out):
    wid = lax.axis_index("s") * NUM_CORES + lax.axis_index("c")
    _extract_one(mtT_hbm, movies_hbm, mout_hbm, NMOVIES,
                 idxbuf_v, cnt_v, cur_v, hits_v, sorted_v, sorted_s, starts_s,
                 blockbuf_v, stage_v, sem_blk, sem_out, wid)
    _extract_one(utT_hbm, users_hbm, uout_hbm, NUSERS,
                 idxbuf_v, cnt_v, cur_v, hits_v, sorted_v, sorted_s, starts_s,
                 blockbuf_v, stage_v, sem_blk, sem_out, wid)


TC_TILE = 2048


def _tc_dot_body(u_ref, m_ref, w_ref, o_ref):
    x = u_ref[...] * m_ref[...]
    o_ref[...] = jax.lax.dot_general(
        x, w_ref[...], (((1,), (1,)), ((), ())),
        preferred_element_type=jnp.float32)


_tc_dot = pl.pallas_call(
    _tc_dot_body,
    grid=(BATCH // TC_TILE,),
    in_specs=[
        pl.BlockSpec((TC_TILE, D), lambda i: (i, 0)),
        pl.BlockSpec((TC_TILE, D), lambda i: (i, 0)),
        pl.BlockSpec((1, D), lambda i: (0, 0)),
    ],
    out_specs=pl.BlockSpec((TC_TILE, 1), lambda i: (i, 0)),
    out_shape=jax.ShapeDtypeStruct((BATCH, 1), jnp.float32),
)


def kernel(users, movies, user_table, movie_table, W):
    urows, mrows = _extract_kernel(users, movies, user_table.T, movie_table.T)
    return _tc_dot(urows.reshape(BATCH, D), mrows.reshape(BATCH, D), W)
